# baseline (speedup 1.0000x reference)
import functools
from typing import Any

import jax
import jax.numpy as jnp
from jax import lax
from jax.experimental import pallas as pl
from jax.experimental.pallas import tpu as pltpu

CHS = [8, 32, 64, 128, 256]
OUT_CH = 36
B = 4
V4, V3, V2, V1 = 642, 2562, 10242, 40962
EPS = 1e-5
TV = 512


def _cdiv(a, b):
    return (a + b - 1) // b


def _norm_lrelu(z, st, gaff, baff, inv_n):
    s1 = st[0, 0:1, :]
    s2 = st[0, 1:2, :]
    m = s1 * inv_n
    var = s2 * inv_n - m * m
    inv = lax.rsqrt(var + EPS)
    zn = (z - m) * (inv * gaff) + baff
    return jnp.where(zn >= 0.0, zn, 0.2 * zn)


def _k1_body(fold, inv_n, *refs):
    if fold:
        x_ref, st_ref, ga_ref, be_ref, w_ref, bup_ref, pm_ref, h_ref, g_ref = refs
        x = _norm_lrelu(x_ref[0], st_ref, ga_ref[...], be_ref[...], inv_n)
    else:
        x_ref, w_ref, bup_ref, pm_ref, h_ref, g_ref = refs
        x = x_ref[0]
    h = lax.dot_general(x, w_ref[...], (((1,), (0,)), ((), ())),
                        preferred_element_type=jnp.float32) + bup_ref[...]
    h_ref[0] = h
    g_ref[0] = lax.dot_general(h, pm_ref[...], (((1,), (0,)), ((), ())),
                               preferred_element_type=jnp.float32)


def _run_k1(xv, stats, gaff, baff, WupT, bup, PM, Vlo, C, inv_n):
    Cin = WupT.shape[0]
    nt = _cdiv(Vlo, TV)
    fold = stats is not None
    ins = [xv]
    specs = [pl.BlockSpec((1, TV, Cin), lambda b, i: (b, i, 0))]
    if fold:
        Cp = stats.shape[2]
        ins += [stats, gaff, baff]
        specs += [pl.BlockSpec((1, 8, Cp), lambda b, i: (b, 0, 0)),
                  pl.BlockSpec((1, Cp), lambda b, i: (0, 0)),
                  pl.BlockSpec((1, Cp), lambda b, i: (0, 0))]
    ins += [WupT, bup, PM]
    specs += [pl.BlockSpec(WupT.shape, lambda b, i: (0, 0)),
              pl.BlockSpec(bup.shape, lambda b, i: (0, 0)),
              pl.BlockSpec(PM.shape, lambda b, i: (0, 0))]
    return pl.pallas_call(
        functools.partial(_k1_body, fold, inv_n),
        grid=(B, nt),
        in_specs=specs,
        out_specs=[pl.BlockSpec((1, TV, 7 * C), lambda b, i: (b, i, 0)),
                   pl.BlockSpec((1, TV, 7 * C // 2), lambda b, i: (b, i, 0))],
        out_shape=[jax.ShapeDtypeStruct((B, Vlo, 7 * C), jnp.float32),
                   jax.ShapeDtypeStruct((B, Vlo, 7 * C // 2), jnp.float32)],
    )(*ins)


def _k3_body(yA_ref, yB_ref, xs_ref, wa_ref, wb_ref, wx_ref, p_ref):
    p = lax.dot_general(yA_ref[0], wa_ref[...], (((1,), (0,)), ((), ())),
                        preferred_element_type=jnp.float32)
    p += lax.dot_general(yB_ref[0], wb_ref[...], (((1,), (0,)), ((), ())),
                         preferred_element_type=jnp.float32)
    p += lax.dot_general(xs_ref[0], wx_ref[...], (((0,), (0,)), ((), ())),
                         preferred_element_type=jnp.float32)
    p_ref[0] = p


def _run_k3(yA, yB, xs, WA, WB, WX, Vhi, C):
    Ch = C // 2
    nt = _cdiv(Vhi, TV)
    return pl.pallas_call(
        _k3_body,
        grid=(B, nt),
        in_specs=[pl.BlockSpec((1, TV, Ch), lambda b, i: (b, i, 0)),
                  pl.BlockSpec((1, TV, Ch), lambda b, i: (b, i, 0)),
                  pl.BlockSpec((1, C, TV), lambda b, i: (b, 0, i)),
                  pl.BlockSpec(WA.shape, lambda b, i: (0, 0)),
                  pl.BlockSpec(WB.shape, lambda b, i: (0, 0)),
                  pl.BlockSpec(WX.shape, lambda b, i: (0, 0))],
        out_specs=pl.BlockSpec((1, TV, 7 * C), lambda b, i: (b, i, 0)),
        out_shape=jax.ShapeDtypeStruct((B, Vhi, 7 * C), jnp.float32),
    )(yA, yB, xs, WA, WB, WX)


def _k5_body(vreal, z_ref, o_ref):
    i = pl.program_id(1)
    rows = lax.broadcasted_iota(jnp.int32, z_ref[0].shape, 0) + i * TV
    z = jnp.where(rows < vreal, z_ref[0], 0.0)
    s1 = jnp.sum(z, axis=0, keepdims=True)
    s2 = jnp.sum(z * z, axis=0, keepdims=True)

    @pl.when(i == 0)
    def _():
        o_ref[0] = jnp.zeros_like(o_ref[0])

    o_ref[0, 0:1, :] += s1
    o_ref[0, 1:2, :] += s2


def _run_k5(z, Vreal, C):
    Vp = z.shape[1]
    nt = _cdiv(Vp, TV)
    return pl.pallas_call(
        functools.partial(_k5_body, Vreal),
        grid=(B, nt),
        in_specs=[pl.BlockSpec((1, TV, C), lambda b, i: (b, i, 0))],
        out_specs=pl.BlockSpec((1, 8, C), lambda b, i: (b, 0, 0)),
        out_shape=jax.ShapeDtypeStruct((B, 8, C), jnp.float32),
    )(z)


def _k6_body(inv_n, z_ref, st_ref, ga_ref, be_ref, w_ref, q_ref):
    zn = _norm_lrelu(z_ref[0], st_ref, ga_ref[...], be_ref[...], inv_n)
    q_ref[0] = lax.dot_general(zn, w_ref[...], (((1,), (0,)), ((), ())),
                               preferred_element_type=jnp.float32)


def _run_k6(z1, stats, gaff, baff, WbT, Vhi, C, inv_n):
    nt = _cdiv(Vhi, TV)
    return pl.pallas_call(
        functools.partial(_k6_body, inv_n),
        grid=(B, nt),
        in_specs=[pl.BlockSpec((1, TV, C), lambda b, i: (b, i, 0)),
                  pl.BlockSpec((1, 8, C), lambda b, i: (b, 0, 0)),
                  pl.BlockSpec((1, C), lambda b, i: (0, 0)),
                  pl.BlockSpec((1, C), lambda b, i: (0, 0)),
                  pl.BlockSpec(WbT.shape, lambda b, i: (0, 0))],
        out_specs=pl.BlockSpec((1, TV, 7 * C), lambda b, i: (b, i, 0)),
        out_shape=jax.ShapeDtypeStruct((B, Vhi, 7 * C), jnp.float32),
    )(z1, stats, gaff, baff, WbT)


def _seg_body(inv_n, z_ref, st_ref, ga_ref, be_ref, w_ref, bs_ref, o_ref):
    zn = _norm_lrelu(z_ref[0], st_ref, ga_ref[...], be_ref[...], inv_n)
    o_ref[0] = lax.dot_general(w_ref[...], zn, (((1,), (1,)), ((), ())),
                               preferred_element_type=jnp.float32) + bs_ref[...]


def _run_seg(z2, stats, gaff, baff, Wseg, bseg, inv_n):
    C = CHS[1]
    nt = _cdiv(V1, TV)
    return pl.pallas_call(
        functools.partial(_seg_body, inv_n),
        grid=(B, nt),
        in_specs=[pl.BlockSpec((1, TV, C), lambda b, i: (b, i, 0)),
                  pl.BlockSpec((1, 8, C), lambda b, i: (b, 0, 0)),
                  pl.BlockSpec((1, C), lambda b, i: (0, 0)),
                  pl.BlockSpec((1, C), lambda b, i: (0, 0)),
                  pl.BlockSpec(Wseg.shape, lambda b, i: (0, 0)),
                  pl.BlockSpec((OUT_CH, 1), lambda b, i: (0, 0))],
        out_specs=pl.BlockSpec((1, OUT_CH, TV), lambda b, i: (b, 0, i)),
        out_shape=jax.ShapeDtypeStruct((B, OUT_CH, V1), jnp.float32),
    )(z2, stats, gaff, baff, Wseg, bseg)


def _gather_y(h, g, top, dA, dB, Vlo, Vhi, C):
    hh = h.reshape(B, 14 * Vlo, C // 2)
    g2 = g.reshape(B, 7 * Vlo, C // 2)
    yA = jnp.concatenate([jnp.take(hh, 2 * top, axis=1),
                          jnp.take(g2, dA, axis=1)], axis=1)
    yB = jnp.concatenate([jnp.take(hh, 2 * top + 1, axis=1),
                          jnp.take(g2, dB, axis=1)], axis=1)
    return yA, yB


def _gather_sum(P, Jk, Vhi, Vp, C):
    Pv = P.reshape(B, 7 * Vhi, C)
    z = sum(jnp.take(Pv, Jk[k], axis=1) for k in range(7))
    return jnp.pad(z, ((0, 0), (0, Vp - Vhi), (0, 0)))


def _prep_w(Wup, Wa, Wb, C):
    WupT = Wup.T
    n = 7 * C // 2
    cols = jnp.arange(n)
    PM = (jnp.zeros((7 * C, n), jnp.float32)
          .at[2 * cols, cols].set(0.5)
          .at[2 * cols + 1, cols].set(0.5))
    WaT = Wa.reshape(C, 7, 2 * C).transpose(2, 1, 0).reshape(2 * C, 7 * C)
    WA, WB, WX = WaT[: C // 2], WaT[C // 2: C], WaT[C:]
    WbT = Wb.reshape(C, 7, C).transpose(2, 1, 0).reshape(C, 7 * C)
    return WupT, PM, WA, WB, WX, WbT


def _prep_J(no, Vhi):
    no2 = no.reshape(Vhi, 7).T * 7 + jnp.arange(7, dtype=jnp.int32)[:, None]
    Vp = 128 * _cdiv(Vhi, 128)
    return jnp.pad(no2, ((0, 0), (0, Vp - Vhi))), Vp


def _block(xv, stats_prev, gprev, bprev, xs, no, top, down,
           Wup, bup, Wa, ga, bea, Wb, gb, beb, Vlo, Vhi, C, inv_n_prev):
    WupT, PM, WA, WB, WX, WbT = _prep_w(Wup, Wa, Wb, C)
    h, g = _run_k1(xv, stats_prev, gprev, bprev, WupT, bup[None, :], PM,
                   Vlo, C, inv_n_prev)
    yA, yB = _gather_y(h, g, top, down[0::2], down[1::2], Vlo, Vhi, C)
    P = _run_k3(yA, yB, xs, WA, WB, WX, Vhi, C)
    Jk, Vp = _prep_J(no, Vhi)
    z1 = _gather_sum(P, Jk, Vhi, Vp, C)
    st1 = _run_k5(z1, Vhi, C)
    Q = _run_k6(z1, st1, ga[None, :], bea[None, :], WbT, Vhi, C, 1.0 / Vhi)
    z2 = _gather_sum(Q, Jk, Vhi, Vp, C)
    st2 = _run_k5(z2, Vhi, C)
    return z2, st2


def kernel(x1, x2, x3, x4, Wup3, bup3, Wa3, ba3, ga3, bea3, Wb3, bb3, gb3, beb3, Wup2, bup2, Wa2, ba2, ga2, bea2, Wb2, bb2, gb2, beb2, Wup1, bup1, Wa1, ba1, ga1, bea1, Wb1, bb1, gb1, beb1, Wseg, bseg, no3, top3, down3, no2, top2, down2, no1, top1, down1):
    xv = jnp.swapaxes(x4, 1, 2)
    z, st = _block(xv, None, None, None, x3, no3, top3, down3,
                   Wup3, bup3, Wa3, ga3, bea3, Wb3, gb3, beb3,
                   V4, V3, CHS[3], None)
    z, st = _block(z, st, gb3[None, :], beb3[None, :], x2, no2, top2, down2,
                   Wup2, bup2, Wa2, ga2, bea2, Wb2, gb2, beb2,
                   V3, V2, CHS[2], 1.0 / V3)
    z, st = _block(z, st, gb2[None, :], beb2[None, :], x1, no1, top1, down1,
                   Wup1, bup1, Wa1, ga1, bea1, Wb1, gb1, beb1,
                   V2, V1, CHS[1], 1.0 / V2)
    return _run_seg(z, st, gb1[None, :], beb1[None, :], Wseg,
                    bseg[:, None], 1.0 / V1)


# speedup vs baseline: 4.9433x; 4.9433x over previous
import functools
from typing import Any

import jax
import jax.numpy as jnp
from jax import lax
from jax.experimental import pallas as pl
from jax.experimental.pallas import tpu as pltpu
from jax.experimental.pallas import tpu_sc as plsc

NC, NS = 2, 16
NW = NC * NS

CHS = [8, 32, 64, 128, 256]
OUT_CH = 36
B = 4
V4, V3, V2, V1 = 642, 2562, 10242, 40962
EPS = 1e-5
TV = 512


def _cdiv(a, b):
    return (a + b - 1) // b


def _norm_lrelu(z, st, gaff, baff, inv_n):
    s1 = st[0, 0:1, :]
    s2 = st[0, 1:2, :]
    m = s1 * inv_n
    var = s2 * inv_n - m * m
    inv = lax.rsqrt(var + EPS)
    zn = (z - m) * (inv * gaff) + baff
    return jnp.where(zn >= 0.0, zn, 0.2 * zn)


def _k1_body(fold, inv_n, *refs):
    if fold:
        x_ref, st_ref, ga_ref, be_ref, w_ref, bup_ref, pm_ref, h_ref = refs
        x = _norm_lrelu(x_ref[0], st_ref, ga_ref[...], be_ref[...], inv_n)
    else:
        x_ref, w_ref, bup_ref, pm_ref, h_ref = refs
        x = x_ref[0]
    h = lax.dot_general(x, w_ref[...], (((1,), (0,)), ((), ())),
                        preferred_element_type=jnp.float32) + bup_ref[...]
    g = lax.dot_general(h, pm_ref[...], (((1,), (0,)), ((), ())),
                        preferred_element_type=jnp.float32)
    h_ref[0] = jnp.concatenate([h, g], axis=1)


def _run_k1(xv, stats, gaff, baff, WupT, bup, PM, Vlo, C, inv_n):
    Cin = WupT.shape[0]
    nt = _cdiv(Vlo, TV)
    fold = stats is not None
    ins = [xv]
    specs = [pl.BlockSpec((1, TV, Cin), lambda b, i: (b, i, 0))]
    if fold:
        Cp = stats.shape[2]
        ins += [stats, gaff, baff]
        specs += [pl.BlockSpec((1, 8, Cp), lambda b, i: (b, 0, 0)),
                  pl.BlockSpec((1, Cp), lambda b, i: (0, 0)),
                  pl.BlockSpec((1, Cp), lambda b, i: (0, 0))]
    ins += [WupT, bup, PM]
    specs += [pl.BlockSpec(WupT.shape, lambda b, i: (0, 0)),
              pl.BlockSpec(bup.shape, lambda b, i: (0, 0)),
              pl.BlockSpec(PM.shape, lambda b, i: (0, 0))]
    W = 21 * C // 2
    return pl.pallas_call(
        functools.partial(_k1_body, fold, inv_n),
        grid=(B, nt),
        in_specs=specs,
        out_specs=pl.BlockSpec((1, TV, W), lambda b, i: (b, i, 0)),
        out_shape=jax.ShapeDtypeStruct((B, Vlo, W), jnp.float32),
    )(*ins)


def _k3_body(yA_ref, yB_ref, xs_ref, wa_ref, wb_ref, wx_ref, p_ref):
    p = lax.dot_general(yA_ref[0], wa_ref[...], (((1,), (0,)), ((), ())),
                        preferred_element_type=jnp.float32)
    p += lax.dot_general(yB_ref[0], wb_ref[...], (((1,), (0,)), ((), ())),
                         preferred_element_type=jnp.float32)
    p += lax.dot_general(xs_ref[0], wx_ref[...], (((0,), (0,)), ((), ())),
                         preferred_element_type=jnp.float32)
    p_ref[0] = p


def _run_k3(yA, yB, xs, WA, WB, WX, Vhi, C):
    Ch = C // 2
    nt = _cdiv(Vhi, TV)
    return pl.pallas_call(
        _k3_body,
        grid=(B, nt),
        in_specs=[pl.BlockSpec((1, TV, Ch), lambda b, i: (b, i, 0)),
                  pl.BlockSpec((1, TV, Ch), lambda b, i: (b, i, 0)),
                  pl.BlockSpec((1, C, TV), lambda b, i: (b, 0, i)),
                  pl.BlockSpec(WA.shape, lambda b, i: (0, 0)),
                  pl.BlockSpec(WB.shape, lambda b, i: (0, 0)),
                  pl.BlockSpec(WX.shape, lambda b, i: (0, 0))],
        out_specs=pl.BlockSpec((1, TV, 7 * C), lambda b, i: (b, i, 0)),
        out_shape=jax.ShapeDtypeStruct((B, Vhi, 7 * C), jnp.float32),
    )(yA, yB, xs, WA, WB, WX)


def _k5_body(vreal, z_ref, o_ref):
    i = pl.program_id(1)
    rows = lax.broadcasted_iota(jnp.int32, z_ref[0].shape, 0) + i * TV
    z = jnp.where(rows < vreal, z_ref[0], 0.0)
    s1 = jnp.sum(z, axis=0, keepdims=True)
    s2 = jnp.sum(z * z, axis=0, keepdims=True)

    @pl.when(i == 0)
    def _():
        o_ref[0] = jnp.zeros_like(o_ref[0])

    o_ref[0, 0:1, :] += s1
    o_ref[0, 1:2, :] += s2


def _run_k5(z, Vreal, C):
    Vp = z.shape[1]
    nt = _cdiv(Vp, TV)
    return pl.pallas_call(
        functools.partial(_k5_body, Vreal),
        grid=(B, nt),
        in_specs=[pl.BlockSpec((1, TV, C), lambda b, i: (b, i, 0))],
        out_specs=pl.BlockSpec((1, 8, C), lambda b, i: (b, 0, 0)),
        out_shape=jax.ShapeDtypeStruct((B, 8, C), jnp.float32),
    )(z)


def _k6_body(inv_n, z_ref, st_ref, ga_ref, be_ref, w_ref, q_ref):
    zn = _norm_lrelu(z_ref[0], st_ref, ga_ref[...], be_ref[...], inv_n)
    q_ref[0] = lax.dot_general(zn, w_ref[...], (((1,), (0,)), ((), ())),
                               preferred_element_type=jnp.float32)


def _run_k6(z1, stats, gaff, baff, WbT, Vhi, C, inv_n):
    nt = _cdiv(Vhi, TV)
    return pl.pallas_call(
        functools.partial(_k6_body, inv_n),
        grid=(B, nt),
        in_specs=[pl.BlockSpec((1, TV, C), lambda b, i: (b, i, 0)),
                  pl.BlockSpec((1, 8, C), lambda b, i: (b, 0, 0)),
                  pl.BlockSpec((1, C), lambda b, i: (0, 0)),
                  pl.BlockSpec((1, C), lambda b, i: (0, 0)),
                  pl.BlockSpec(WbT.shape, lambda b, i: (0, 0))],
        out_specs=pl.BlockSpec((1, TV, 7 * C), lambda b, i: (b, i, 0)),
        out_shape=jax.ShapeDtypeStruct((B, Vhi, 7 * C), jnp.float32),
    )(z1, stats, gaff, baff, WbT)


def _seg_body(inv_n, z_ref, st_ref, ga_ref, be_ref, w_ref, bs_ref, o_ref):
    zn = _norm_lrelu(z_ref[0], st_ref, ga_ref[...], be_ref[...], inv_n)
    o_ref[0] = lax.dot_general(w_ref[...], zn, (((1,), (1,)), ((), ())),
                               preferred_element_type=jnp.float32) + bs_ref[...]


def _run_seg(z2, stats, gaff, baff, Wseg, bseg, inv_n):
    C = CHS[1]
    nt = _cdiv(V1, TV)
    return pl.pallas_call(
        functools.partial(_seg_body, inv_n),
        grid=(B, nt),
        in_specs=[pl.BlockSpec((1, TV, C), lambda b, i: (b, i, 0)),
                  pl.BlockSpec((1, 8, C), lambda b, i: (b, 0, 0)),
                  pl.BlockSpec((1, C), lambda b, i: (0, 0)),
                  pl.BlockSpec((1, C), lambda b, i: (0, 0)),
                  pl.BlockSpec(Wseg.shape, lambda b, i: (0, 0)),
                  pl.BlockSpec((OUT_CH, 1), lambda b, i: (0, 0))],
        out_specs=pl.BlockSpec((1, OUT_CH, TV), lambda b, i: (b, 0, i)),
        out_shape=jax.ShapeDtypeStruct((B, OUT_CH, V1), jnp.float32),
    )(z2, stats, gaff, baff, Wseg, bseg)


@functools.lru_cache(maxsize=1)
def _mesh():
    return plsc.VectorSubcoreMesh(core_axis_name="c", subcore_axis_name="s",
                                  num_cores=NC, num_subcores=NS)


def _wid():
    return lax.axis_index("s") * NC + lax.axis_index("c")


def _sc_build_y(Uf, IA, IB, Vhi, C):
    Ch = C // 2
    T = _cdiv(Vhi, 128)
    YP = T * 128
    tw = -(-T // NW)
    out = jax.ShapeDtypeStruct((B, YP, Ch), jnp.float32)

    @functools.partial(
        pl.kernel,
        out_type=(out, out),
        mesh=_mesh(),
        compiler_params=pltpu.CompilerParams(use_tc_tiling_on_sc=False),
        scratch_types=[
            pltpu.VMEM((tw, 128), jnp.int32),
            pltpu.VMEM((tw, 128), jnp.int32),
            pltpu.VMEM((128, Ch), jnp.float32),
            pltpu.VMEM((128, Ch), jnp.float32),
            pltpu.SemaphoreType.DMA,
        ],
    )
    def k(u_ref, ia_ref, ib_ref, ya_ref, yb_ref, jva, jvb, bufa, bufb, sem):
        w = _wid()
        for b in range(B):
            pltpu.sync_copy(ia_ref.at[b, pl.ds(w * tw, tw)], jva)
            pltpu.sync_copy(ib_ref.at[b, pl.ds(w * tw, tw)], jvb)

            def cb(ti, _):
                t = w * tw + ti

                @pl.when(t < T)
                def _():
                    ca = pltpu.make_async_copy(u_ref.at[jva.at[ti]], bufa, sem)
                    cb2 = pltpu.make_async_copy(u_ref.at[jvb.at[ti]], bufb, sem)
                    ca.start()
                    cb2.start()
                    ca.wait()
                    cb2.wait()
                    pltpu.sync_copy(bufa, ya_ref.at[b, pl.ds(t * 128, 128)])
                    pltpu.sync_copy(bufb, yb_ref.at[b, pl.ds(t * 128, 128)])
                return 0

            lax.fori_loop(0, tw, cb, 0)

    return k(Uf, IA, IB)


def _sc_gather_sum(Pv, J4, Vhi, C, R, nch, nchw):
    Vp = nch * R

    @functools.partial(
        pl.kernel,
        out_type=jax.ShapeDtypeStruct((B, Vp, C), jnp.float32),
        mesh=_mesh(),
        compiler_params=pltpu.CompilerParams(use_tc_tiling_on_sc=False),
        scratch_types=[
            pltpu.VMEM((nchw, 7, R), jnp.int32),
            pltpu.VMEM((7, R, C), jnp.float32),
            pltpu.VMEM((R, C), jnp.float32),
            pltpu.SemaphoreType.DMA,
        ],
    )
    def k(pv_ref, j_ref, z_ref, jv, gb, acc, sem):
        w = _wid()
        for b in range(B):
            pltpu.sync_copy(j_ref.at[b, pl.ds(w * nchw, nchw)], jv)

            def cb(ci, _):
                c = w * nchw + ci

                @pl.when(c < nch)
                def _():
                    cps = [pltpu.make_async_copy(pv_ref.at[jv.at[ci, k]],
                                                 gb.at[k], sem)
                           for k in range(7)]
                    for cp in cps:
                        cp.start()
                    for cp in cps:
                        cp.wait()

                    def rb(r, _2):
                        for j in range(C // 16):
                            sl = pl.ds(j * 16, 16)
                            v = gb[0, r, sl]
                            for kk in range(1, 7):
                                v = v + gb[kk, r, sl]
                            acc[r, sl] = v
                        return 0

                    lax.fori_loop(0, R, rb, 0)
                    pltpu.sync_copy(acc, z_ref.at[b, pl.ds(c * R, R)])
                return 0

            lax.fori_loop(0, nchw, cb, 0)

    return k(Pv, J4)


def _prep_w(Wup, Wa, Wb, C):
    WupT = Wup.T
    n = 7 * C // 2
    cols = jnp.arange(n)
    PM = (jnp.zeros((7 * C, n), jnp.float32)
          .at[2 * cols, cols].set(0.5)
          .at[2 * cols + 1, cols].set(0.5))
    WaT = Wa.reshape(C, 7, 2 * C).transpose(2, 1, 0).reshape(2 * C, 7 * C)
    WA, WB, WX = WaT[: C // 2], WaT[C // 2: C], WaT[C:]
    WbT = Wb.reshape(C, 7, C).transpose(2, 1, 0).reshape(C, 7 * C)
    return WupT, PM, WA, WB, WX, WbT


def _prep_y_idx(top, down, Vlo, Vhi):
    T = _cdiv(Vhi, 128)
    Tp = NW * (-(-T // NW))
    iA = 21 * (top // 7) + 2 * (top % 7)
    dA, dB = down[0::2], down[1::2]
    dAu = 21 * (dA // 7) + 14 + (dA % 7)
    dBu = 21 * (dB // 7) + 14 + (dB % 7)
    padn = Tp * 128 - Vhi
    IA = jnp.pad(jnp.concatenate([dAu, iA]), (0, padn)).reshape(Tp, 128)
    IB = jnp.pad(jnp.concatenate([dBu, iA + 1]), (0, padn)).reshape(Tp, 128)
    off = (jnp.arange(B, dtype=jnp.int32) * (21 * Vlo))[:, None, None]
    return IA[None] + off, IB[None] + off


def _prep_gs_idx(no, Vhi, C, Vlo=None, YP=None):
    R = 64 if C == 128 else 128
    nch = -(-Vhi // R)
    nchw = -(-nch // NW)
    Vp = nch * R
    no2 = no.reshape(Vhi, 7).T
    if Vlo is not None:
        N2 = Vhi - Vlo
        no2 = jnp.where(no2 < Vlo, N2 + no2, no2 - Vlo)
        rows = YP
    else:
        rows = Vhi
    no2 = no2 * 7 + jnp.arange(7, dtype=jnp.int32)[:, None]
    JT = (jnp.pad(no2, ((0, 0), (0, Vp - Vhi)))
          .reshape(7, nch, R).transpose(1, 0, 2))
    JT = jnp.pad(JT, ((0, NW * nchw - nch), (0, 0), (0, 0)))
    J4 = JT[None] + (jnp.arange(B, dtype=jnp.int32)
                     * (7 * rows))[:, None, None, None]
    return J4, R, nch, nchw


def _block(xv, stats_prev, gprev, bprev, xs, no, top, down,
           Wup, bup, Wa, ga, bea, Wb, gb, beb, Vlo, Vhi, C, inv_n_prev):
    WupT, PM, WA, WB, WX, WbT = _prep_w(Wup, Wa, Wb, C)
    U = _run_k1(xv, stats_prev, gprev, bprev, WupT, bup[None, :], PM,
                Vlo, C, inv_n_prev)
    IA, IB = _prep_y_idx(top, down, Vlo, Vhi)
    yA, yB = _sc_build_y(U.reshape(B * 21 * Vlo, C // 2), IA, IB, Vhi, C)
    YP = yA.shape[1]
    xsP = jnp.concatenate([xs[:, :, Vlo:], xs[:, :, :Vlo]], axis=2)
    P = _run_k3(yA, yB, xsP, WA, WB, WX, YP, C)
    J4p, R, nch, nchw = _prep_gs_idx(no, Vhi, C, Vlo=Vlo, YP=YP)
    z1 = _sc_gather_sum(P.reshape(B * 7 * YP, C), J4p, Vhi, C, R, nch, nchw)
    st1 = _run_k5(z1, Vhi, C)
    Q = _run_k6(z1, st1, ga[None, :], bea[None, :], WbT, Vhi, C, 1.0 / Vhi)
    J4, R, nch, nchw = _prep_gs_idx(no, Vhi, C)
    z2 = _sc_gather_sum(Q.reshape(B * 7 * Vhi, C), J4, Vhi, C, R, nch, nchw)
    st2 = _run_k5(z2, Vhi, C)
    return z2, st2


def kernel(x1, x2, x3, x4, Wup3, bup3, Wa3, ba3, ga3, bea3, Wb3, bb3, gb3, beb3, Wup2, bup2, Wa2, ba2, ga2, bea2, Wb2, bb2, gb2, beb2, Wup1, bup1, Wa1, ba1, ga1, bea1, Wb1, bb1, gb1, beb1, Wseg, bseg, no3, top3, down3, no2, top2, down2, no1, top1, down1):
    xv = jnp.swapaxes(x4, 1, 2)
    z, st = _block(xv, None, None, None, x3, no3, top3, down3,
                   Wup3, bup3, Wa3, ga3, bea3, Wb3, gb3, beb3,
                   V4, V3, CHS[3], None)
    z, st = _block(z, st, gb3[None, :], beb3[None, :], x2, no2, top2, down2,
                   Wup2, bup2, Wa2, ga2, bea2, Wb2, gb2, beb2,
                   V3, V2, CHS[2], 1.0 / V3)
    z, st = _block(z, st, gb2[None, :], beb2[None, :], x1, no1, top1, down1,
                   Wup1, bup1, Wa1, ga1, bea1, Wb1, gb1, beb1,
                   V2, V1, CHS[1], 1.0 / V2)
    return _run_seg(z, st, gb1[None, :], beb1[None, :], Wseg,
                    bseg[:, None], 1.0 / V1)
